# baseline (device time: 30389 ns/iter reference)
import jax
import jax.numpy as jnp
from jax import lax
from jax.experimental import pallas as pl
from jax.experimental.pallas import tpu as pltpu

Y = 4


def kernel(x):
    m_per, n_tot = x.shape
    n_per = n_tot // Y

    def body(x_ref, out_ref, xblk, comm, rbuf, fetch_sems, wb_sems, send_sems, recv_sems):
        ix = lax.axis_index("x")
        iy = lax.axis_index("y")
        iz = lax.axis_index("z")

        fetches = []
        for k in range(Y):
            dst = (iy + k) % Y
            cp = pltpu.make_async_copy(
                x_ref.at[:, pl.ds(dst * n_per, n_per)],
                xblk.at[(k - 1) % Y],
                fetch_sems.at[(k - 1) % Y],
            )
            cp.start()
            fetches.append(cp)

        barrier_sem = pltpu.get_barrier_semaphore()
        for k in range(1, Y):
            peer = (iy + k) % Y
            pl.semaphore_signal(
                barrier_sem, inc=1,
                device_id=(ix, peer, iz),
                device_id_type=pl.DeviceIdType.MESH,
            )
        pl.semaphore_wait(barrier_sem, Y - 1)

        rdmas = []
        for k in range(1, Y):
            dst = (iy + k) % Y
            fetches[k].wait()
            comm[k - 1, :, :] = xblk[k - 1, :, :].astype(comm.dtype)
            rdma = pltpu.make_async_remote_copy(
                src_ref=comm.at[k - 1],
                dst_ref=rbuf.at[k - 1],
                send_sem=send_sems.at[k - 1],
                recv_sem=recv_sems.at[k - 1],
                device_id=(ix, dst, iz),
                device_id_type=pl.DeviceIdType.MESH,
            )
            rdma.start()
            rdmas.append(rdma)

        fetches[0].wait()
        comm[Y - 1, :, :] = xblk[Y - 1, :, :].astype(comm.dtype)
        wbs = [
            pltpu.make_async_copy(
                comm.at[Y - 1],
                out_ref.at[pl.ds(iy * m_per, m_per), :],
                wb_sems.at[Y - 1],
            )
        ]
        wbs[0].start()

        for k in range(1, Y):
            src = (iy - k) % Y
            recv = pltpu.make_async_remote_copy(
                src_ref=comm.at[k - 1],
                dst_ref=rbuf.at[k - 1],
                send_sem=send_sems.at[k - 1],
                recv_sem=recv_sems.at[k - 1],
                device_id=(ix, src, iz),
                device_id_type=pl.DeviceIdType.MESH,
            )
            recv.wait_recv()
            wb = pltpu.make_async_copy(
                rbuf.at[k - 1],
                out_ref.at[pl.ds(src * m_per, m_per), :],
                wb_sems.at[k - 1],
            )
            wb.start()
            wbs.append(wb)

        for wb in wbs:
            wb.wait()
        for rdma in rdmas:
            rdma.wait_send()

    out_shape = jax.ShapeDtypeStruct((Y * m_per, n_per), jnp.bfloat16)
    return pl.pallas_call(
        body,
        out_shape=out_shape,
        in_specs=[pl.BlockSpec(memory_space=pl.ANY)],
        out_specs=pl.BlockSpec(memory_space=pl.ANY),
        scratch_shapes=[
            pltpu.VMEM((Y, m_per, n_per), jnp.float32),
            pltpu.VMEM((Y, m_per, n_per), jnp.bfloat16),
            pltpu.VMEM((Y - 1, m_per, n_per), jnp.bfloat16),
            pltpu.SemaphoreType.DMA((Y,)),
            pltpu.SemaphoreType.DMA((Y,)),
            pltpu.SemaphoreType.DMA((Y - 1,)),
            pltpu.SemaphoreType.DMA((Y - 1,)),
        ],
        compiler_params=pltpu.CompilerParams(collective_id=0),
    )(x)


# device time: 29515 ns/iter; 1.0296x vs baseline; 1.0296x over previous
import jax
import jax.numpy as jnp
from jax import lax
from jax.experimental import pallas as pl
from jax.experimental.pallas import tpu as pltpu

Y = 4


def kernel(x):
    m_per, n_tot = x.shape
    n_per = n_tot // Y

    def body(x_ref, out_ref, comm_ref, send_sems, recv_sems):
        ix = lax.axis_index("x")
        iy = lax.axis_index("y")
        iz = lax.axis_index("z")

        barrier_sem = pltpu.get_barrier_semaphore()
        for k in range(1, Y):
            peer = (iy + k) % Y
            pl.semaphore_signal(
                barrier_sem, inc=1,
                device_id=(ix, peer, iz),
                device_id_type=pl.DeviceIdType.MESH,
            )

        for k in range(1, Y):
            dst = (iy + k) % Y
            comm_ref[k - 1, :, :] = x_ref[:, pl.ds(dst * n_per, n_per)].astype(
                comm_ref.dtype
            )
        out_ref[pl.ds(iy * m_per, m_per), :] = x_ref[
            :, pl.ds(iy * n_per, n_per)
        ].astype(out_ref.dtype)

        pl.semaphore_wait(barrier_sem, Y - 1)

        rdmas = []
        for k in range(1, Y):
            dst = (iy + k) % Y
            rdma = pltpu.make_async_remote_copy(
                src_ref=comm_ref.at[k - 1],
                dst_ref=out_ref.at[pl.ds(iy * m_per, m_per), :],
                send_sem=send_sems.at[k - 1],
                recv_sem=recv_sems.at[k - 1],
                device_id=(ix, dst, iz),
                device_id_type=pl.DeviceIdType.MESH,
            )
            rdma.start()
            rdmas.append(rdma)

        for k in range(1, Y):
            src = (iy - k) % Y
            recv = pltpu.make_async_remote_copy(
                src_ref=comm_ref.at[k - 1],
                dst_ref=out_ref.at[pl.ds(src * m_per, m_per), :],
                send_sem=send_sems.at[k - 1],
                recv_sem=recv_sems.at[k - 1],
                device_id=(ix, src, iz),
                device_id_type=pl.DeviceIdType.MESH,
            )
            recv.wait_recv()

        for rdma in rdmas:
            rdma.wait_send()

    out_shape = jax.ShapeDtypeStruct((Y * m_per, n_per), jnp.bfloat16)
    return pl.pallas_call(
        body,
        out_shape=out_shape,
        in_specs=[pl.BlockSpec(memory_space=pltpu.VMEM)],
        out_specs=pl.BlockSpec(memory_space=pltpu.VMEM),
        scratch_shapes=[
            pltpu.VMEM((Y - 1, m_per, n_per), jnp.bfloat16),
            pltpu.SemaphoreType.DMA((Y - 1,)),
            pltpu.SemaphoreType.DMA((Y - 1,)),
        ],
        compiler_params=pltpu.CompilerParams(collective_id=0),
    )(x)
